# baseline (speedup 1.0000x reference)
import functools

import jax
import jax.numpy as jnp
from jax import lax
from jax.experimental import pallas as pl
from jax.experimental.pallas import tpu as pltpu
from jax.experimental.pallas import tpu_sc as plsc

N_DOM = 26
N_CONC = 100000
P_DIM = 16
BATCH_B = 16384

NW = 32
LANES = 16

M_TILES_FULL = N_CONC // 128
M_TAIL = N_CONC - M_TILES_FULL * 128
TILES_PER_DOM = M_TILES_FULL + 1
G_TILES = N_DOM * TILES_PER_DOM
T_BASE = G_TILES // NW
T_REM = G_TILES - T_BASE * NW

IDX_ROWS = BATCH_B * N_DOM // 128
B_TILES = BATCH_B // 128
CHUNKS_PER_W = IDX_ROWS // NW
LAB_SLAB = BATCH_B // LANES


def _mesh():
    return plsc.VectorSubcoreMesh(core_axis_name="c", subcore_axis_name="s")


def _prep_kernel(tabT, labelsT, tailT):
    @functools.partial(
        pl.kernel,
        mesh=_mesh(),
        out_type=(
            jax.ShapeDtypeStruct((N_DOM * N_CONC, P_DIM), jnp.float32),
            jax.ShapeDtypeStruct((IDX_ROWS, 128), jnp.int32),
        ),
        scratch_types=[
            pltpu.VMEM((P_DIM, 128), jnp.float32),
            pltpu.VMEM((128, P_DIM), jnp.float32),
            pltpu.VMEM((N_DOM, LAB_SLAB), jnp.int32),
            pltpu.VMEM((8, 128), jnp.int32),
        ],
        compiler_params=pltpu.CompilerParams(
            use_tc_tiling_on_sc=True, needs_layout_passes=False),
    )
    def k(tabT_hbm, labT_hbm, tailT_hbm, tabR_hbm, idxR_hbm, va, tr, lab, w8):
        wid = lax.axis_index("s") * 2 + lax.axis_index("c")
        lanes = lax.iota(jnp.int32, LANES)
        rowv = [lanes + g * LANES for g in range(8)]
        pcon = [lanes * 0 + p for p in range(P_DIM)]

        def tile_body(i, carry):
            t = wid + NW * i
            d = t // TILES_PER_DOM
            kk = t % TILES_PER_DOM
            m0 = pl.multiple_of(kk * 128, 128)

            @pl.when(kk < M_TILES_FULL)
            def _():
                pltpu.sync_copy(
                    tabT_hbm.at[d, pl.ds(0, P_DIM), pl.ds(m0, 128)], va)

            @pl.when(kk == M_TILES_FULL)
            def _():
                pltpu.sync_copy(tailT_hbm.at[d], va)

            for p in range(P_DIM):
                for g in range(8):
                    v = va[p, pl.ds(g * LANES, LANES)]
                    plsc.store_scatter(tr, [rowv[g], pcon[p]], v)

            r0 = pl.multiple_of(d * N_CONC + m0, 8)

            @pl.when(kk < M_TILES_FULL)
            def _():
                pltpu.sync_copy(tr, tabR_hbm.at[pl.ds(r0, 128)])

            @pl.when(kk == M_TILES_FULL)
            def _():
                pltpu.sync_copy(tr.at[pl.ds(0, M_TAIL)],
                                tabR_hbm.at[pl.ds(r0, M_TAIL)])

            return carry

        n_tiles = T_BASE + (wid < T_REM).astype(jnp.int32)
        lax.fori_loop(0, n_tiles, tile_body, 0)

        b0 = pl.multiple_of(wid * LAB_SLAB, 128)
        pltpu.sync_copy(labT_hbm.at[pl.ds(0, N_DOM), pl.ds(b0, LAB_SLAB)],
                        lab)

        def dom_body(dd, carry):
            off = dd * N_CONC
            for jj in range(8):
                for g in range(8):
                    src = (dd, pl.ds(jj * 128 + g * LANES, LANES))
                    w8[jj, pl.ds(g * LANES, LANES)] = lab[src] + off
            q0 = pl.multiple_of(dd * B_TILES + wid * 8, 8)
            pltpu.sync_copy(w8, idxR_hbm.at[pl.ds(q0, 8)])
            return carry

        lax.fori_loop(0, N_DOM, dom_body, 0)

    return k(tabT, labelsT, tailT)


def _gather_kernel(tabR, idxR):
    @functools.partial(
        pl.kernel,
        mesh=_mesh(),
        out_type=jax.ShapeDtypeStruct((N_DOM, 2, B_TILES, 8, 128),
                                      jnp.float32),
        scratch_types=[
            pltpu.VMEM((128,), jnp.int32),
            pltpu.VMEM((128, P_DIM), jnp.float32),
            pltpu.VMEM((2, 8, 128), jnp.float32),
            pltpu.SemaphoreType.DMA,
        ],
        compiler_params=pltpu.CompilerParams(
            use_tc_tiling_on_sc=False, needs_layout_passes=False),
    )
    def k(tabR_hbm, idxR_hbm, out_hbm, iv, rows, ov, sem):
        wid = lax.axis_index("s") * 2 + lax.axis_index("c")
        lanes = lax.iota(jnp.int32, LANES)
        rowv = [lanes + g * LANES for g in range(8)]
        pcon = [lanes * 0 + p for p in range(P_DIM)]

        def chunk_body(i, carry):
            r = wid * CHUNKS_PER_W + i
            d = r // B_TILES
            bt = r % B_TILES
            pltpu.sync_copy(idxR_hbm.at[r], iv)
            pltpu.async_copy(tabR_hbm.at[iv], rows, sem).wait()
            for p in range(P_DIM):
                for g in range(8):
                    col = plsc.load_gather(rows, [rowv[g], pcon[p]])
                    ov[p // 8, p % 8, pl.ds(g * LANES, LANES)] = col
            pltpu.sync_copy(ov, out_hbm.at[d, pl.ds(0, 2), bt])
            return carry

        lax.fori_loop(0, CHUNKS_PER_W, chunk_body, 0)

    return k(tabR, idxR)


def kernel(labels, pqc_params):
    labelsT = labels.astype(jnp.int32).T
    tabT = jnp.transpose(pqc_params, (0, 2, 1))
    tailT = jnp.pad(tabT[:, :, M_TILES_FULL * 128:],
                    ((0, 0), (0, 0), (0, 128 - M_TAIL)))
    tabR, idxR = _prep_kernel(tabT, labelsT, tailT)
    out5 = _gather_kernel(tabR, idxR)
    out = jnp.transpose(out5, (2, 4, 0, 1, 3))
    return out.reshape(BATCH_B, N_DOM, P_DIM)


# speedup vs baseline: 1.5753x; 1.5753x over previous
import functools

import jax
import jax.numpy as jnp
from jax import lax
from jax.experimental import pallas as pl
from jax.experimental.pallas import tpu as pltpu
from jax.experimental.pallas import tpu_sc as plsc

N_DOM = 26
N_CONC = 100000
P_DIM = 16
BATCH_B = 16384

NW = 32
LANES = 16

M_TILES_FULL = N_CONC // 128
M_TAIL = N_CONC - M_TILES_FULL * 128
TILES_PER_DOM = M_TILES_FULL + 1
M_PAD = TILES_PER_DOM * 128
G_TILES = N_DOM * TILES_PER_DOM
T_PER_W = -(-G_TILES // NW)
DUMMY_R0 = N_DOM * M_PAD
TAB_ROWS = DUMMY_R0 + NW * 128

IDX_ROWS = BATCH_B * N_DOM // 128
B_TILES = BATCH_B // 128
CHUNKS_PER_W = IDX_ROWS // NW
LAB_W = 16
LAB_SLAB = BATCH_B // LAB_W


def _mesh():
    return plsc.VectorSubcoreMesh(core_axis_name="c", subcore_axis_name="s")


def _prep_kernel(tabT, labelsT, tailT):
    @functools.partial(
        pl.kernel,
        mesh=_mesh(),
        out_type=(
            jax.ShapeDtypeStruct((TAB_ROWS, P_DIM), jnp.float32),
            jax.ShapeDtypeStruct((IDX_ROWS, 128), jnp.int32),
        ),
        scratch_types=[
            pltpu.VMEM((2, P_DIM, 128), jnp.float32),
            pltpu.VMEM((2, 128, P_DIM), jnp.float32),
            pltpu.VMEM((N_DOM, LAB_SLAB), jnp.int32),
            pltpu.VMEM((8, 128), jnp.int32),
            pltpu.SemaphoreType.DMA,
            pltpu.SemaphoreType.DMA,
            pltpu.SemaphoreType.DMA,
            pltpu.SemaphoreType.DMA,
        ],
        compiler_params=pltpu.CompilerParams(
            use_tc_tiling_on_sc=True, needs_layout_passes=False),
    )
    def k(tabT_hbm, labT_hbm, tailT_hbm, tabR_hbm, idxR_hbm,
          va, tr, lab, w8, si0, si1, so0, so1):
        wid = lax.axis_index("s") * 2 + lax.axis_index("c")
        lanes = lax.iota(jnp.int32, LANES)
        rowv = [lanes + g * LANES for g in range(8)]
        pcon = [lanes * 0 + p for p in range(P_DIM)]
        sin = (si0, si1)
        sout = (so0, so1)

        def tile_coords(i):
            t = wid + NW * i
            tc = lax.min(t, G_TILES - 1)
            d = tc // TILES_PER_DOM
            kk = tc % TILES_PER_DOM
            m0 = pl.multiple_of(kk * 128, 128)
            r0 = lax.select(t < G_TILES, d * M_PAD + m0,
                            jnp.int32(DUMMY_R0) + wid * 128)
            return d, kk, m0, pl.multiple_of(r0, 8)

        def start_in(i, b):
            d, kk, m0, _ = tile_coords(i)

            @pl.when(kk < M_TILES_FULL)
            def _():
                pltpu.async_copy(
                    tabT_hbm.at[d, pl.ds(0, P_DIM), pl.ds(m0, 128)],
                    va.at[b], sin[b])

            @pl.when(kk == M_TILES_FULL)
            def _():
                pltpu.async_copy(tailT_hbm.at[d], va.at[b], sin[b])

        def wait_in(b):
            pltpu.make_async_copy(
                tabT_hbm.at[0, pl.ds(0, P_DIM), pl.ds(0, 128)],
                va.at[b], sin[b]).wait()

        def wait_out(b):
            pltpu.make_async_copy(
                tr.at[b], tabR_hbm.at[pl.ds(DUMMY_R0, 128)], sout[b]).wait()

        start_in(0, 0)
        start_in(1, 1)

        def pair_body(j, carry):
            for b in range(2):
                i = 2 * j + b
                _, _, _, r0 = tile_coords(i)
                wait_in(b)

                @pl.when(j >= 1)
                def _():
                    wait_out(b)

                for p in range(P_DIM):
                    for g in range(8):
                        v = va[b, p, pl.ds(g * LANES, LANES)]
                        plsc.store_scatter(tr.at[b], [rowv[g], pcon[p]], v)
                pltpu.async_copy(tr.at[b], tabR_hbm.at[pl.ds(r0, 128)],
                                 sout[b])

                @pl.when(j < T_PER_W // 2 - 1)
                def _():
                    start_in(i + 2, b)
            return carry

        lax.fori_loop(0, T_PER_W // 2, pair_body, 0)
        wait_out(0)
        wait_out(1)

        @pl.when(wid < LAB_W)
        def _():
            b0 = pl.multiple_of(wid * LAB_SLAB, 128)
            pltpu.sync_copy(
                labT_hbm.at[pl.ds(0, N_DOM), pl.ds(b0, LAB_SLAB)], lab)

            def dom_body(dd, carry):
                off = dd * M_PAD
                for jj in range(8):
                    for g in range(8):
                        src = (dd, pl.ds(jj * 128 + g * LANES, LANES))
                        w8[jj, pl.ds(g * LANES, LANES)] = lab[src] + off
                q0 = pl.multiple_of(dd * B_TILES + wid * 8, 8)
                pltpu.sync_copy(w8, idxR_hbm.at[pl.ds(q0, 8)])
                return carry

            lax.fori_loop(0, N_DOM, dom_body, 0)

    return k(tabT, labelsT, tailT)


def _gather_kernel(tabR, idxR):
    @functools.partial(
        pl.kernel,
        mesh=_mesh(),
        out_type=jax.ShapeDtypeStruct((N_DOM, 2, B_TILES, 8, 128),
                                      jnp.float32),
        scratch_types=[
            pltpu.VMEM((2, 128), jnp.int32),
            pltpu.VMEM((2, 128, P_DIM), jnp.float32),
            pltpu.VMEM((2, 2, 8, 128), jnp.float32),
            pltpu.SemaphoreType.DMA,
            pltpu.SemaphoreType.DMA,
            pltpu.SemaphoreType.DMA,
            pltpu.SemaphoreType.DMA,
            pltpu.SemaphoreType.DMA,
            pltpu.SemaphoreType.DMA,
        ],
        compiler_params=pltpu.CompilerParams(
            use_tc_tiling_on_sc=False, needs_layout_passes=False),
    )
    def k(tabR_hbm, idxR_hbm, out_hbm, iv, rows, ov,
          sx0, sx1, sg0, sg1, so0, so1):
        wid = lax.axis_index("s") * 2 + lax.axis_index("c")
        lanes = lax.iota(jnp.int32, LANES)
        rowv = [lanes + g * LANES for g in range(8)]
        pcon = [lanes * 0 + p for p in range(P_DIM)]
        sx = (sx0, sx1)
        sg = (sg0, sg1)
        so = (so0, so1)

        def start_idx(c, b):
            pltpu.async_copy(idxR_hbm.at[wid * CHUNKS_PER_W + c],
                             iv.at[b], sx[b])

        def wait_idx(b):
            pltpu.make_async_copy(idxR_hbm.at[0], iv.at[b], sx[b]).wait()

        def wait_gather(b):
            pltpu.make_async_copy(tabR_hbm.at[pl.ds(0, 128)], rows.at[b],
                                  sg[b]).wait()

        def wait_out(b):
            pltpu.make_async_copy(
                ov.at[b], out_hbm.at[0, pl.ds(0, 2), 0], so[b]).wait()

        def compute_store(c, b):
            r = wid * CHUNKS_PER_W + c
            d = r // B_TILES
            bt = r % B_TILES
            for p in range(P_DIM):
                for g in range(8):
                    col = plsc.load_gather(rows.at[b], [rowv[g], pcon[p]])
                    ov[b, p // 8, p % 8, pl.ds(g * LANES, LANES)] = col
            pltpu.async_copy(ov.at[b], out_hbm.at[d, pl.ds(0, 2), bt], so[b])

        start_idx(0, 0)

        def pair_body(j, carry):
            for b in range(2):
                c = 2 * j + b
                wait_idx(b)
                pltpu.async_copy(tabR_hbm.at[iv.at[b]], rows.at[b], sg[b])
                if b == 0:
                    @pl.when(j >= 1)
                    def _():
                        wait_gather(1)

                    @pl.when(j >= 2)
                    def _():
                        wait_out(1)

                    @pl.when(j >= 1)
                    def _():
                        compute_store(c - 1, 1)
                else:
                    wait_gather(0)

                    @pl.when(j >= 1)
                    def _():
                        wait_out(0)
                    compute_store(c - 1, 0)

                @pl.when(c + 1 < CHUNKS_PER_W)
                def _():
                    start_idx(c + 1, 1 - b)
            return carry

        lax.fori_loop(0, CHUNKS_PER_W // 2, pair_body, 0)
        wait_gather(1)
        wait_out(1)
        compute_store(CHUNKS_PER_W - 1, 1)
        wait_out(0)
        wait_out(1)

    return k(tabR, idxR)


def kernel(labels, pqc_params):
    labelsT = labels.astype(jnp.int32).T
    tabT = jnp.transpose(pqc_params, (0, 2, 1))
    tailT = jnp.pad(tabT[:, :, M_TILES_FULL * 128:],
                    ((0, 0), (0, 0), (0, 128 - M_TAIL)))
    tabR, idxR = _prep_kernel(tabT, labelsT, tailT)
    out5 = _gather_kernel(tabR, idxR)
    out = jnp.transpose(out5, (2, 4, 0, 1, 3))
    return out.reshape(BATCH_B, N_DOM, P_DIM)


# speedup vs baseline: 4.6412x; 2.9462x over previous
import functools

import jax
import jax.numpy as jnp
from jax import lax
from jax.experimental import pallas as pl
from jax.experimental.pallas import tpu as pltpu
from jax.experimental.pallas import tpu_sc as plsc

N_DOM = 26
N_CONC = 100000
P_DIM = 16
BATCH_B = 16384

NW = 32
LANES = 16

M_TILES_FULL = N_CONC // 128
M_TAIL = N_CONC - M_TILES_FULL * 128
TILES_PER_DOM = M_TILES_FULL + 1
M_PAD = TILES_PER_DOM * 128
SLAB_M = 256
SLABS_PER_DOM = M_PAD // SLAB_M
G_SLABS = N_DOM * SLABS_PER_DOM
S_PER_W = -(-G_SLABS // NW)
R128_PER_DOM = M_PAD * P_DIM // 128
DUMMY_R128 = N_DOM * R128_PER_DOM
TAB_R128 = DUMMY_R128 + NW * 32

IDX_ROWS = BATCH_B * N_DOM // 128
B_TILES = BATCH_B // 128
CHUNKS_PER_W = IDX_ROWS // NW
LAB_W = 16
LAB_SLAB = BATCH_B // LAB_W


def _mesh():
    return plsc.VectorSubcoreMesh(core_axis_name="c", subcore_axis_name="s")


def _prep_kernel(tabT, labelsT, tailT):
    @functools.partial(
        pl.kernel,
        mesh=_mesh(),
        out_type=(
            jax.ShapeDtypeStruct((TAB_R128, 128), jnp.float32),
            jax.ShapeDtypeStruct((IDX_ROWS, 128), jnp.int32),
        ),
        scratch_types=[
            pltpu.VMEM((2, P_DIM, SLAB_M), jnp.float32),
            pltpu.VMEM((2, 32, 128), jnp.float32),
            pltpu.VMEM((N_DOM, LAB_SLAB), jnp.int32),
            pltpu.VMEM((8, 128), jnp.int32),
            pltpu.SemaphoreType.DMA,
            pltpu.SemaphoreType.DMA,
            pltpu.SemaphoreType.DMA,
            pltpu.SemaphoreType.DMA,
        ],
        compiler_params=pltpu.CompilerParams(
            use_tc_tiling_on_sc=True, needs_layout_passes=False),
    )
    def k(tabT_hbm, labT_hbm, tailT_hbm, tabR_hbm, idxR_hbm,
          va, tr, lab, w8, si0, si1, so0, so1):
        wid = lax.axis_index("s") * 2 + lax.axis_index("c")
        lanes = lax.iota(jnp.int32, LANES)
        rbase = lax.shift_right_logical(lanes, 3)
        rowv = [rbase + 2 * g for g in range(SLAB_M // LANES)]
        ccon = [(lanes & 7) * LANES + p for p in range(P_DIM)]
        sin = (si0, si1)
        sout = (so0, so1)

        def slab_coords(i):
            t = wid + NW * i
            tc = lax.min(t, G_SLABS - 1)
            d = tc // SLABS_PER_DOM
            ks = tc % SLABS_PER_DOM
            m0 = pl.multiple_of(ks * SLAB_M, 128)
            r0 = lax.select(t < G_SLABS, d * R128_PER_DOM + ks * 32,
                            jnp.int32(DUMMY_R128) + wid * 32)
            return d, ks, m0, pl.multiple_of(r0, 8)

        def start_in(i, b):
            d, ks, m0, _ = slab_coords(i)

            @pl.when(ks < SLABS_PER_DOM - 1)
            def _():
                pltpu.async_copy(
                    tabT_hbm.at[d, pl.ds(0, P_DIM), pl.ds(m0, SLAB_M)],
                    va.at[b], sin[b])

            @pl.when(ks == SLABS_PER_DOM - 1)
            def _():
                pltpu.async_copy(
                    tabT_hbm.at[d, pl.ds(0, P_DIM),
                                pl.ds(M_TILES_FULL * 128 - 128, 128)],
                    va.at[b, pl.ds(0, P_DIM), pl.ds(0, 128)], sin[b])
                pltpu.async_copy(
                    tailT_hbm.at[d],
                    va.at[b, pl.ds(0, P_DIM), pl.ds(128, 128)], sin[b])

        def wait_in(b):
            pltpu.make_async_copy(
                tabT_hbm.at[0, pl.ds(0, P_DIM), pl.ds(0, SLAB_M)],
                va.at[b], sin[b]).wait()

        def wait_out(b):
            pltpu.make_async_copy(
                tr.at[b], tabR_hbm.at[pl.ds(DUMMY_R128, 32)], sout[b]).wait()

        start_in(0, 0)
        start_in(1, 1)

        def pair_body(j, carry):
            for b in range(2):
                i = 2 * j + b
                _, _, _, r0 = slab_coords(i)
                wait_in(b)

                @pl.when(j >= 1)
                def _():
                    wait_out(b)

                for p in range(P_DIM):
                    for g in range(SLAB_M // LANES):
                        v = va[b, p, pl.ds(g * LANES, LANES)]
                        plsc.store_scatter(tr.at[b], [rowv[g], ccon[p]], v)
                pltpu.async_copy(tr.at[b], tabR_hbm.at[pl.ds(r0, 32)],
                                 sout[b])

                @pl.when(j < S_PER_W // 2 - 1)
                def _():
                    start_in(i + 2, b)
            return carry

        lax.fori_loop(0, S_PER_W // 2, pair_body, 0)
        wait_out(0)
        wait_out(1)

        @pl.when(wid < LAB_W)
        def _():
            b0 = pl.multiple_of(wid * LAB_SLAB, 128)
            pltpu.sync_copy(
                labT_hbm.at[pl.ds(0, N_DOM), pl.ds(b0, LAB_SLAB)], lab)

            def dom_body(dd, carry):
                off = dd * M_PAD
                for jj in range(8):
                    for g in range(8):
                        src = (dd, pl.ds(jj * 128 + g * LANES, LANES))
                        w8[jj, pl.ds(g * LANES, LANES)] = lab[src] + off
                q0 = pl.multiple_of(dd * B_TILES + wid * 8, 8)
                pltpu.sync_copy(w8, idxR_hbm.at[pl.ds(q0, 8)])
                return carry

            lax.fori_loop(0, N_DOM, dom_body, 0)

    return k(tabT, labelsT, tailT)


def _gather_kernel(tabR, idxR):
    @functools.partial(
        pl.kernel,
        mesh=_mesh(),
        out_type=jax.ShapeDtypeStruct((N_DOM, 2, B_TILES, 8, 128),
                                      jnp.float32),
        scratch_types=[
            pltpu.VMEM((2, 128), jnp.int32),
            pltpu.VMEM((2, 128, P_DIM), jnp.float32),
            pltpu.VMEM((2, 2, 8, 128), jnp.float32),
            pltpu.SemaphoreType.DMA,
            pltpu.SemaphoreType.DMA,
            pltpu.SemaphoreType.DMA,
            pltpu.SemaphoreType.DMA,
            pltpu.SemaphoreType.DMA,
            pltpu.SemaphoreType.DMA,
        ],
        compiler_params=pltpu.CompilerParams(
            use_tc_tiling_on_sc=False, needs_layout_passes=False),
    )
    def k(tabR_hbm, idxR_hbm, out_hbm, iv, rows, ov,
          sx0, sx1, sg0, sg1, so0, so1):
        wid = lax.axis_index("s") * 2 + lax.axis_index("c")
        lanes = lax.iota(jnp.int32, LANES)
        rowv = [lanes + g * LANES for g in range(8)]
        pcon = [lanes * 0 + p for p in range(P_DIM)]
        sx = (sx0, sx1)
        sg = (sg0, sg1)
        so = (so0, so1)

        def start_idx(c, b):
            pltpu.async_copy(idxR_hbm.at[wid * CHUNKS_PER_W + c],
                             iv.at[b], sx[b])

        def wait_idx(b):
            pltpu.make_async_copy(idxR_hbm.at[0], iv.at[b], sx[b]).wait()

        def wait_gather(b):
            pltpu.make_async_copy(tabR_hbm.at[pl.ds(0, 128)], rows.at[b],
                                  sg[b]).wait()

        def wait_out(b):
            pltpu.make_async_copy(
                ov.at[b], out_hbm.at[0, pl.ds(0, 2), 0], so[b]).wait()

        def compute_store(c, b):
            r = wid * CHUNKS_PER_W + c
            d = r // B_TILES
            bt = r % B_TILES
            for p in range(P_DIM):
                for g in range(8):
                    col = plsc.load_gather(rows.at[b], [rowv[g], pcon[p]])
                    ov[b, p // 8, p % 8, pl.ds(g * LANES, LANES)] = col
            pltpu.async_copy(ov.at[b], out_hbm.at[d, pl.ds(0, 2), bt], so[b])

        start_idx(0, 0)

        def pair_body(j, carry):
            for b in range(2):
                c = 2 * j + b
                wait_idx(b)
                pltpu.async_copy(tabR_hbm.at[iv.at[b]], rows.at[b], sg[b])
                if b == 0:
                    @pl.when(j >= 1)
                    def _():
                        wait_gather(1)

                    @pl.when(j >= 2)
                    def _():
                        wait_out(1)

                    @pl.when(j >= 1)
                    def _():
                        compute_store(c - 1, 1)
                else:
                    wait_gather(0)

                    @pl.when(j >= 1)
                    def _():
                        wait_out(0)
                    compute_store(c - 1, 0)

                @pl.when(c + 1 < CHUNKS_PER_W)
                def _():
                    start_idx(c + 1, 1 - b)
            return carry

        lax.fori_loop(0, CHUNKS_PER_W // 2, pair_body, 0)
        wait_gather(1)
        wait_out(1)
        compute_store(CHUNKS_PER_W - 1, 1)
        wait_out(0)
        wait_out(1)

    return k(tabR, idxR)


def kernel(labels, pqc_params):
    labelsT = labels.astype(jnp.int32).T
    tabT = jnp.transpose(pqc_params, (0, 2, 1))
    tailT = jnp.pad(tabT[:, :, M_TILES_FULL * 128:],
                    ((0, 0), (0, 0), (0, 128 - M_TAIL)))
    tabR128, idxR = _prep_kernel(tabT, labelsT, tailT)
    out5 = _gather_kernel(tabR128.reshape(-1, P_DIM), idxR)
    out = jnp.transpose(out5, (2, 4, 0, 1, 3))
    return out.reshape(BATCH_B, N_DOM, P_DIM)


# speedup vs baseline: 4.7770x; 1.0293x over previous
import functools

import jax
import jax.numpy as jnp
from jax import lax
from jax.experimental import pallas as pl
from jax.experimental.pallas import tpu as pltpu
from jax.experimental.pallas import tpu_sc as plsc

N_DOM = 26
N_CONC = 100000
P_DIM = 16
BATCH_B = 16384

NW = 32
LANES = 16

M_TILES_FULL = N_CONC // 128
M_TAIL = N_CONC - M_TILES_FULL * 128
TILES_PER_DOM = M_TILES_FULL + 1
M_PAD = TILES_PER_DOM * 128
SLAB_M = 256
SLABS_PER_DOM = M_PAD // SLAB_M
G_SLABS = N_DOM * SLABS_PER_DOM
S_PER_W = -(-G_SLABS // NW)
R128_PER_DOM = M_PAD * P_DIM // 128
DUMMY_R128 = N_DOM * R128_PER_DOM
TAB_R128 = DUMMY_R128 + NW * 32

IDX_ROWS = BATCH_B * N_DOM // 128
B_TILES = BATCH_B // 128
CHUNKS_PER_W = IDX_ROWS // NW
LAB_W = 16
LAB_SLAB = BATCH_B // LAB_W


def _mesh():
    return plsc.VectorSubcoreMesh(core_axis_name="c", subcore_axis_name="s")


def _prep_kernel(tabT, labelsT, tailT):
    @functools.partial(
        pl.kernel,
        mesh=_mesh(),
        out_type=(
            jax.ShapeDtypeStruct((TAB_R128, 128), jnp.float32),
            jax.ShapeDtypeStruct((IDX_ROWS, 128), jnp.int32),
        ),
        scratch_types=[
            pltpu.VMEM((2, P_DIM, SLAB_M), jnp.float32),
            pltpu.VMEM((2, 32, 128), jnp.float32),
            pltpu.VMEM((N_DOM, LAB_SLAB), jnp.int32),
            pltpu.VMEM((8, 128), jnp.int32),
            pltpu.SemaphoreType.DMA,
            pltpu.SemaphoreType.DMA,
            pltpu.SemaphoreType.DMA,
            pltpu.SemaphoreType.DMA,
        ],
        compiler_params=pltpu.CompilerParams(
            use_tc_tiling_on_sc=True, needs_layout_passes=False),
    )
    def k(tabT_hbm, labT_hbm, tailT_hbm, tabR_hbm, idxR_hbm,
          va, tr, lab, w8, si0, si1, so0, so1):
        wid = lax.axis_index("s") * 2 + lax.axis_index("c")
        lanes = lax.iota(jnp.int32, LANES)
        rbase = lax.shift_right_logical(lanes, 3)
        rowv = [rbase + 2 * g for g in range(SLAB_M // LANES)]
        ccon = [(lanes & 7) * LANES + p for p in range(P_DIM)]
        sin = (si0, si1)
        sout = (so0, so1)

        def slab_coords(i):
            t = wid + NW * i
            tc = lax.min(t, G_SLABS - 1)
            d = tc // SLABS_PER_DOM
            ks = tc % SLABS_PER_DOM
            m0 = pl.multiple_of(ks * SLAB_M, 128)
            r0 = lax.select(t < G_SLABS, d * R128_PER_DOM + ks * 32,
                            jnp.int32(DUMMY_R128) + wid * 32)
            return d, ks, m0, pl.multiple_of(r0, 8)

        def start_in(i, b):
            d, ks, m0, _ = slab_coords(i)

            @pl.when(ks < SLABS_PER_DOM - 1)
            def _():
                pltpu.async_copy(
                    tabT_hbm.at[d, pl.ds(0, P_DIM), pl.ds(m0, SLAB_M)],
                    va.at[b], sin[b])

            @pl.when(ks == SLABS_PER_DOM - 1)
            def _():
                pltpu.async_copy(
                    tabT_hbm.at[d, pl.ds(0, P_DIM),
                                pl.ds(M_TILES_FULL * 128 - 128, 128)],
                    va.at[b, pl.ds(0, P_DIM), pl.ds(0, 128)], sin[b])
                pltpu.async_copy(
                    tailT_hbm.at[d],
                    va.at[b, pl.ds(0, P_DIM), pl.ds(128, 128)], sin[b])

        def wait_in(b):
            pltpu.make_async_copy(
                tabT_hbm.at[0, pl.ds(0, P_DIM), pl.ds(0, SLAB_M)],
                va.at[b], sin[b]).wait()

        def wait_out(b):
            pltpu.make_async_copy(
                tr.at[b], tabR_hbm.at[pl.ds(DUMMY_R128, 32)], sout[b]).wait()

        start_in(0, 0)
        start_in(1, 1)

        def pair_body(j, carry):
            for b in range(2):
                i = 2 * j + b
                _, _, _, r0 = slab_coords(i)
                wait_in(b)

                @pl.when(j >= 1)
                def _():
                    wait_out(b)

                for p in range(P_DIM):
                    for g in range(SLAB_M // LANES):
                        v = va[b, p, pl.ds(g * LANES, LANES)]
                        plsc.store_scatter(tr.at[b], [rowv[g], ccon[p]], v)
                pltpu.async_copy(tr.at[b], tabR_hbm.at[pl.ds(r0, 32)],
                                 sout[b])

                @pl.when(j < S_PER_W // 2 - 1)
                def _():
                    start_in(i + 2, b)
            return carry

        lax.fori_loop(0, S_PER_W // 2, pair_body, 0)
        wait_out(0)
        wait_out(1)

        @pl.when(wid < LAB_W)
        def _():
            b0 = pl.multiple_of(wid * LAB_SLAB, 128)
            pltpu.sync_copy(
                labT_hbm.at[pl.ds(0, N_DOM), pl.ds(b0, LAB_SLAB)], lab)

            def dom_body(dd, carry):
                off = dd * M_PAD
                for jj in range(8):
                    for g in range(8):
                        src = (dd, pl.ds(jj * 128 + g * LANES, LANES))
                        w8[jj, pl.ds(g * LANES, LANES)] = lab[src] + off
                q0 = pl.multiple_of(dd * B_TILES + wid * 8, 8)
                pltpu.sync_copy(w8, idxR_hbm.at[pl.ds(q0, 8)])
                return carry

            lax.fori_loop(0, N_DOM, dom_body, 0)

    return k(tabT, labelsT, tailT)


def _gather_kernel(tabR, idxR):
    @functools.partial(
        pl.kernel,
        mesh=_mesh(),
        out_type=jax.ShapeDtypeStruct((N_DOM, 2, B_TILES, 8, 128),
                                      jnp.float32),
        scratch_types=[
            pltpu.VMEM((2, 2, 128), jnp.int32),
            pltpu.VMEM((2, 256, P_DIM), jnp.float32),
            pltpu.VMEM((2, 2, 2, 8, 128), jnp.float32),
            pltpu.SemaphoreType.DMA,
            pltpu.SemaphoreType.DMA,
            pltpu.SemaphoreType.DMA,
            pltpu.SemaphoreType.DMA,
            pltpu.SemaphoreType.DMA,
            pltpu.SemaphoreType.DMA,
        ],
        compiler_params=pltpu.CompilerParams(
            use_tc_tiling_on_sc=False, needs_layout_passes=False),
    )
    def k(tabR_hbm, idxR_hbm, out_hbm, iv, rows, ov,
          sx0, sx1, sg0, sg1, so0, so1):
        wid = lax.axis_index("s") * 2 + lax.axis_index("c")
        lanes = lax.iota(jnp.int32, LANES)
        rowv = [lanes + g * LANES for g in range(16)]
        pcon = [lanes * 0 + p for p in range(P_DIM)]
        sx = (sx0, sx1)
        sg = (sg0, sg1)
        so = (so0, so1)

        def start_idx(c, b):
            pltpu.async_copy(idxR_hbm.at[pl.ds(wid * CHUNKS_PER_W + 2 * c, 2)],
                             iv.at[b], sx[b])

        def wait_idx(b):
            pltpu.make_async_copy(idxR_hbm.at[pl.ds(0, 2)], iv.at[b],
                                  sx[b]).wait()

        def wait_gather(b):
            pltpu.make_async_copy(tabR_hbm.at[pl.ds(0, 256)], rows.at[b],
                                  sg[b]).wait()

        def wait_out(b):
            pltpu.make_async_copy(
                ov.at[b], out_hbm.at[0, pl.ds(0, 2), pl.ds(0, 2)],
                so[b]).wait()

        def compute_store(c, b):
            r = wid * CHUNKS_PER_W + 2 * c
            d = r // B_TILES
            bt = pl.multiple_of(r % B_TILES, 2)
            for h in range(2):
                for p in range(P_DIM):
                    for g in range(8):
                        col = plsc.load_gather(
                            rows.at[b], [rowv[h * 8 + g], pcon[p]])
                        ov[b, p // 8, h, p % 8,
                           pl.ds(g * LANES, LANES)] = col
            pltpu.async_copy(
                ov.at[b], out_hbm.at[d, pl.ds(0, 2), pl.ds(bt, 2)], so[b])

        start_idx(0, 0)

        def pair_body(j, carry):
            for b in range(2):
                c = 2 * j + b
                wait_idx(b)
                pltpu.async_copy(tabR_hbm.at[iv.at[b, 0]],
                                 rows.at[b, pl.ds(0, 128)], sg[b])
                pltpu.async_copy(tabR_hbm.at[iv.at[b, 1]],
                                 rows.at[b, pl.ds(128, 128)], sg[b])
                if b == 0:
                    @pl.when(j >= 1)
                    def _():
                        wait_gather(1)

                    @pl.when(j >= 2)
                    def _():
                        wait_out(1)

                    @pl.when(j >= 1)
                    def _():
                        compute_store(c - 1, 1)
                else:
                    wait_gather(0)

                    @pl.when(j >= 1)
                    def _():
                        wait_out(0)
                    compute_store(c - 1, 0)

                @pl.when(c + 1 < CHUNKS_PER_W // 2)
                def _():
                    start_idx(c + 1, 1 - b)
            return carry

        lax.fori_loop(0, CHUNKS_PER_W // 4, pair_body, 0)
        wait_gather(1)
        wait_out(1)
        compute_store(CHUNKS_PER_W // 2 - 1, 1)
        wait_out(0)
        wait_out(1)

    return k(tabR, idxR)


def kernel(labels, pqc_params):
    labelsT = labels.astype(jnp.int32).T
    tabT = jnp.transpose(pqc_params, (0, 2, 1))
    tailT = jnp.pad(tabT[:, :, M_TILES_FULL * 128:],
                    ((0, 0), (0, 0), (0, 128 - M_TAIL)))
    tabR128, idxR = _prep_kernel(tabT, labelsT, tailT)
    out5 = _gather_kernel(tabR128.reshape(-1, P_DIM), idxR)
    out = jnp.transpose(out5, (2, 4, 0, 1, 3))
    return out.reshape(BATCH_B, N_DOM, P_DIM)
